# baseline (speedup 1.0000x reference)
import jax
import jax.numpy as jnp
from jax.experimental import pallas as pl


def _add_kernel(x_ref, e_ref, o_ref):
    o_ref[...] = x_ref[...] + e_ref[...][None]


def kernel(x, embedding):
    B, T, D = x.shape
    TT = 1024
    grid = (T // TT, B)
    return pl.pallas_call(
        _add_kernel,
        grid=grid,
        in_specs=[
            pl.BlockSpec((1, TT, D), lambda i, b: (b, i, 0)),
            pl.BlockSpec((TT, D), lambda i, b: (i, 0)),
        ],
        out_specs=pl.BlockSpec((1, TT, D), lambda i, b: (b, i, 0)),
        out_shape=jax.ShapeDtypeStruct((B, T, D), x.dtype),
    )(x, embedding)
